# baseline (speedup 1.0000x reference)
import jax
import jax.numpy as jnp
from jax import lax
from jax.experimental import pallas as pl
from jax.experimental.pallas import tpu as pltpu

_NCB = 4
_K = 512
_D = 64
_TT = 1152
_A = 128

_HI = lax.Precision.HIGHEST
_BF = jnp.bfloat16


def _split3(x):
    hi = x.astype(_BF)
    r1 = x - hi.astype(jnp.float32)
    mid = r1.astype(_BF)
    lo = (r1 - mid.astype(jnp.float32)).astype(_BF)
    return hi, mid, lo


def _dot(a, b, dims):
    return lax.dot_general(a, b, (dims, ((), ())),
                           preferred_element_type=jnp.float32)


def _onehot_dot3(oh, parts, dims):
    h, m, l = parts
    ohb = oh.astype(_BF)
    return (_dot(ohb, h, dims) + _dot(ohb, m, dims)) + _dot(ohb, l, dims)


def _exact_dist(rt, wt, tb):
    diff = rt[:, :, None] - wt[:, None, :]
    sq = diff * diff
    x = sq.reshape(8, 8, tb, _K)
    x = x[:, 0:4] + x[:, 4:8]
    x = x[:, 0:2] + x[:, 2:4]
    x = x[:, 0] + x[:, 1]
    d = x[0]
    for g in range(1, 8):
        d = d + x[g]
    return d


def _first_min_idx(d, iota_k):
    m = jnp.min(d, axis=1, keepdims=True)
    return jnp.min(jnp.where(d == m, iota_k, _K), axis=1, keepdims=True), m


def _rvq_block(z_ref, cbt_ref, qsum_ref, inds_ref, loss_ref, idx_scr, zq_scr):
    i = pl.program_id(0)
    r = z_ref[...]
    lane_k = lax.broadcasted_iota(jnp.int32, (_TT, _K), 1)
    lane_ka = lax.broadcasted_iota(jnp.int32, (_A, _K), 1)
    lane128 = lax.broadcasted_iota(jnp.int32, (_TT, 128), 1)
    lane_a = lax.broadcasted_iota(jnp.int32, (_TT, _A), 1)
    row_t = lax.broadcasted_iota(jnp.int32, (_TT, _TT), 0)
    col_t = lax.broadcasted_iota(jnp.int32, (_TT, _TT), 1)
    ltri = (col_t < row_t).astype(jnp.float32)
    ind_tile = jnp.zeros((_TT, 128), jnp.int32)
    qacc = jnp.zeros((_TT, _D), jnp.float32)
    loss_val = jnp.float32(0.0)
    for c in range(_NCB):
        wt = cbt_ref[c]
        wparts = _split3(wt)
        rparts = _split3(r)
        s2 = jnp.sum(r * r, axis=1, keepdims=True)
        w2 = jnp.sum(wt * wt, axis=0, keepdims=True)
        cdims = ((1,), (0,))
        m2 = (_dot(rparts[0], wparts[0], cdims)
              + _dot(rparts[1], wparts[0], cdims)
              + _dot(rparts[0], wparts[1], cdims))
        shat = (s2 + w2) - (m2 + m2)
        idx, mhat = _first_min_idx(shat, lane_k)
        delta = s2 * jnp.float32(8e-6) + jnp.float32(4e-5)
        cnt = jnp.sum((shat <= mhat + delta).astype(jnp.int32),
                      axis=1, keepdims=True)
        amb = cnt > 1
        ambf = amb.astype(jnp.float32)
        namb = jnp.sum(ambf)
        rank = lax.dot_general(ltri, ambf, (((1,), (0,)), ((), ())),
                               preferred_element_type=jnp.float32)
        idx_scr[...] = jnp.broadcast_to(idx, (_TT, 128))
        oh = (lane_k == idx).astype(jnp.float32)
        zq0 = _onehot_dot3(oh, wparts, ((1,), (1,)))

        def _chunk(lo, wt=wt, wparts=wparts, rparts=rparts):
            in_rng = amb & (rank >= lo) & (rank < lo + _A)
            slot = (rank - lo).astype(jnp.int32)
            pt = (in_rng & (lane_a == slot)).astype(jnp.float32)
            rc = _onehot_dot3(pt, rparts, ((0,), (0,)))
            dex = _exact_dist(rc.T, wt, _A)
            iex, _ = _first_min_idx(dex, lane_ka)
            ohc = (lane_ka == iex).astype(jnp.float32)
            zqc = _onehot_dot3(ohc, wparts, ((1,), (1,)))
            zqf = _onehot_dot3(pt, _split3(zqc), ((1,), (0,)))
            iexf = iex.astype(jnp.float32)
            ih = iexf.astype(_BF)
            il = (iexf - ih.astype(jnp.float32)).astype(_BF)
            ptb = pt.astype(_BF)
            scat = (_dot(ptb, ih, ((1,), (0,)))
                    + _dot(ptb, il, ((1,), (0,))))
            return in_rng, zqf, scat

        in0, zqf0, scat0 = _chunk(jnp.float32(0.0))
        zq_scr[...] = jnp.where(jnp.broadcast_to(in0, (_TT, _D)), zqf0, zq0)
        idx_scr[...] = jnp.where(jnp.broadcast_to(in0, (_TT, 128)),
                                 jnp.broadcast_to(scat0.astype(jnp.int32),
                                                  (_TT, 128)),
                                 idx_scr[...])

        @pl.when(namb > jnp.float32(_A) + 0.5)
        def _(chunk=_chunk):
            nch = (namb.astype(jnp.int32) + (_A - 1)) // _A

            def _ovf(j, carry):
                in_j, zqf_j, scat_j = chunk((j * _A).astype(jnp.float32))
                zq_scr[...] = jnp.where(jnp.broadcast_to(in_j, (_TT, _D)),
                                        zqf_j, zq_scr[...])
                idx_scr[...] = jnp.where(
                    jnp.broadcast_to(in_j, (_TT, 128)),
                    jnp.broadcast_to(scat_j.astype(jnp.int32), (_TT, 128)),
                    idx_scr[...])
                return carry

            lax.fori_loop(1, nch, _ovf, 0)

        zq = zq_scr[...]
        idxf = idx_scr[:, 0:1]
        t = zq - r
        u = r + t
        loss_val = loss_val + jnp.sum(t * t)
        qacc = qacc + u
        r = r - u
        ind_tile = jnp.where(lane128 == c,
                             jnp.broadcast_to(idxf, (_TT, 128)), ind_tile)
    qsum_ref[...] = qacc
    inds_ref[...] = ind_tile

    @pl.when(i == 0)
    def _():
        loss_ref[...] = jnp.zeros_like(loss_ref)

    loss_ref[...] += jnp.full((8, 128), loss_val, jnp.float32)


def kernel(z, codebooks):
    B, N, D = z.shape
    T = B * N
    zf = z.reshape(T, D)
    cbt = jnp.transpose(codebooks, (0, 2, 1))
    qsum, indsw, lossw = pl.pallas_call(
        _rvq_block,
        grid=(T // _TT,),
        in_specs=[
            pl.BlockSpec((_TT, _D), lambda i: (i, 0)),
            pl.BlockSpec((_NCB, _D, _K), lambda i: (0, 0, 0)),
        ],
        out_specs=[
            pl.BlockSpec((_TT, _D), lambda i: (i, 0)),
            pl.BlockSpec((_TT, 128), lambda i: (i, 0)),
            pl.BlockSpec((8, 128), lambda i: (0, 0)),
        ],
        out_shape=[
            jax.ShapeDtypeStruct((T, _D), jnp.float32),
            jax.ShapeDtypeStruct((T, 128), jnp.int32),
            jax.ShapeDtypeStruct((8, 128), jnp.float32),
        ],
        scratch_shapes=[pltpu.VMEM((_TT, 128), jnp.int32),
                        pltpu.VMEM((_TT, _D), jnp.float32)],
    )(zf, cbt)
    quant_sum = qsum.reshape(B, N, D)
    inds = indsw[:, :_NCB].reshape(B, N, _NCB).transpose(0, 2, 1)
    total_loss = lossw[0, 0] * jnp.float32(2.0 / (B * N * D))
    return quant_sum, inds, total_loss


# speedup vs baseline: 1.3526x; 1.3526x over previous
import jax
import jax.numpy as jnp
from jax import lax
from jax.experimental import pallas as pl
from jax.experimental.pallas import tpu as pltpu

_NCB = 4
_K = 512
_D = 64
_TT = 576
_A = 48

_HI = lax.Precision.HIGHEST
_BF = jnp.bfloat16


def _split3(x):
    hi = x.astype(_BF)
    r1 = x - hi.astype(jnp.float32)
    mid = r1.astype(_BF)
    lo = (r1 - mid.astype(jnp.float32)).astype(_BF)
    return hi, mid, lo


def _dot(a, b, dims):
    return lax.dot_general(a, b, (dims, ((), ())),
                           preferred_element_type=jnp.float32)


def _onehot_dot3(oh, parts, dims):
    h, m, l = parts
    ohb = oh.astype(_BF)
    return (_dot(ohb, h, dims) + _dot(ohb, m, dims)) + _dot(ohb, l, dims)


def _exact_dist(rt, wt, tb):
    diff = rt[:, :, None] - wt[:, None, :]
    sq = diff * diff
    x = sq.reshape(8, 8, tb, _K)
    x = x[:, 0:4] + x[:, 4:8]
    x = x[:, 0:2] + x[:, 2:4]
    x = x[:, 0] + x[:, 1]
    d = x[0]
    for g in range(1, 8):
        d = d + x[g]
    return d


def _first_min_idx(d, iota_k):
    m = jnp.min(d, axis=1, keepdims=True)
    return jnp.min(jnp.where(d == m, iota_k, _K), axis=1, keepdims=True), m


def _rvq_block(z_ref, cbt_ref, qsum_ref, inds_ref, loss_ref, idx_scr, zq_scr):
    i = pl.program_id(0)
    r = z_ref[...]
    lane_k = lax.broadcasted_iota(jnp.int32, (_TT, _K), 1)
    lane_ka = lax.broadcasted_iota(jnp.int32, (_A, _K), 1)
    lane128 = lax.broadcasted_iota(jnp.int32, (_TT, 128), 1)
    lane_a = lax.broadcasted_iota(jnp.int32, (_TT, _A), 1)
    row_t = lax.broadcasted_iota(jnp.int32, (_TT, _TT), 0)
    col_t = lax.broadcasted_iota(jnp.int32, (_TT, _TT), 1)
    ltri = (col_t < row_t).astype(jnp.float32)
    ind_tile = jnp.zeros((_TT, 128), jnp.int32)
    qacc = jnp.zeros((_TT, _D), jnp.float32)
    loss_val = jnp.float32(0.0)
    for c in range(_NCB):
        wt = cbt_ref[c]
        wparts = _split3(wt)
        rparts = _split3(r)
        s2 = jnp.sum(r * r, axis=1, keepdims=True)
        w2 = jnp.sum(wt * wt, axis=0, keepdims=True)
        cdims = ((1,), (0,))
        m2 = (_dot(rparts[0], wparts[0], cdims)
              + _dot(rparts[1], wparts[0], cdims)
              + _dot(rparts[0], wparts[1], cdims))
        shat = (s2 + w2) - (m2 + m2)
        idx, mhat = _first_min_idx(shat, lane_k)
        delta = s2 * jnp.float32(6e-6) + jnp.float32(3e-5)
        cnt = jnp.sum((shat <= mhat + delta).astype(jnp.int32),
                      axis=1, keepdims=True)
        amb = cnt > 1
        ambf = amb.astype(jnp.float32)
        namb = jnp.sum(ambf)
        rank = lax.dot_general(ltri, ambf, (((1,), (0,)), ((), ())),
                               preferred_element_type=jnp.float32)
        idx_scr[...] = jnp.broadcast_to(idx, (_TT, 128))
        oh = (lane_k == idx).astype(jnp.float32)
        zq0 = _onehot_dot3(oh, wparts, ((1,), (1,)))

        def _chunk(lo, wt=wt, wparts=wparts, rparts=rparts):
            in_rng = amb & (rank >= lo) & (rank < lo + _A)
            slot = (rank - lo).astype(jnp.int32)
            pt = (in_rng & (lane_a == slot)).astype(jnp.float32)
            rc = _onehot_dot3(pt, rparts, ((0,), (0,)))
            dex = _exact_dist(rc.T, wt, _A)
            iex, _ = _first_min_idx(dex, lane_ka)
            ohc = (lane_ka == iex).astype(jnp.float32)
            zqc = _onehot_dot3(ohc, wparts, ((1,), (1,)))
            zqf = _onehot_dot3(pt, _split3(zqc), ((1,), (0,)))
            iexf = iex.astype(jnp.float32)
            ih = iexf.astype(_BF)
            il = (iexf - ih.astype(jnp.float32)).astype(_BF)
            ptb = pt.astype(_BF)
            scat = (_dot(ptb, ih, ((1,), (0,)))
                    + _dot(ptb, il, ((1,), (0,))))
            return in_rng, zqf, scat

        in0, zqf0, scat0 = _chunk(jnp.float32(0.0))
        zq_scr[...] = jnp.where(jnp.broadcast_to(in0, (_TT, _D)), zqf0, zq0)
        idx_scr[...] = jnp.where(jnp.broadcast_to(in0, (_TT, 128)),
                                 jnp.broadcast_to(scat0.astype(jnp.int32),
                                                  (_TT, 128)),
                                 idx_scr[...])

        @pl.when(namb > jnp.float32(_A) + 0.5)
        def _(chunk=_chunk):
            nch = (namb.astype(jnp.int32) + (_A - 1)) // _A

            def _ovf(j, carry):
                in_j, zqf_j, scat_j = chunk((j * _A).astype(jnp.float32))
                zq_scr[...] = jnp.where(jnp.broadcast_to(in_j, (_TT, _D)),
                                        zqf_j, zq_scr[...])
                idx_scr[...] = jnp.where(
                    jnp.broadcast_to(in_j, (_TT, 128)),
                    jnp.broadcast_to(scat_j.astype(jnp.int32), (_TT, 128)),
                    idx_scr[...])
                return carry

            lax.fori_loop(1, nch, _ovf, 0)

        zq = zq_scr[...]
        idxf = idx_scr[:, 0:1]
        t = zq - r
        u = r + t
        loss_val = loss_val + jnp.sum(t * t)
        qacc = qacc + u
        r = r - u
        ind_tile = jnp.where(lane128 == c,
                             jnp.broadcast_to(idxf, (_TT, 128)), ind_tile)
    qsum_ref[...] = qacc
    inds_ref[...] = ind_tile

    @pl.when(i == 0)
    def _():
        loss_ref[...] = jnp.zeros_like(loss_ref)

    loss_ref[...] += jnp.full((8, 128), loss_val, jnp.float32)


def kernel(z, codebooks):
    B, N, D = z.shape
    T = B * N
    zf = z.reshape(T, D)
    cbt = jnp.transpose(codebooks, (0, 2, 1))
    qsum, indsw, lossw = pl.pallas_call(
        _rvq_block,
        grid=(T // _TT,),
        in_specs=[
            pl.BlockSpec((_TT, _D), lambda i: (i, 0)),
            pl.BlockSpec((_NCB, _D, _K), lambda i: (0, 0, 0)),
        ],
        out_specs=[
            pl.BlockSpec((_TT, _D), lambda i: (i, 0)),
            pl.BlockSpec((_TT, 128), lambda i: (i, 0)),
            pl.BlockSpec((8, 128), lambda i: (0, 0)),
        ],
        out_shape=[
            jax.ShapeDtypeStruct((T, _D), jnp.float32),
            jax.ShapeDtypeStruct((T, 128), jnp.int32),
            jax.ShapeDtypeStruct((8, 128), jnp.float32),
        ],
        scratch_shapes=[pltpu.VMEM((_TT, 128), jnp.int32),
                        pltpu.VMEM((_TT, _D), jnp.float32)],
    )(zf, cbt)
    quant_sum = qsum.reshape(B, N, D)
    inds = indsw[:, :_NCB].reshape(B, N, _NCB).transpose(0, 2, 1)
    total_loss = lossw[0, 0] * jnp.float32(2.0 / (B * N * D))
    return quant_sum, inds, total_loss


# speedup vs baseline: 1.4775x; 1.0923x over previous
import jax
import jax.numpy as jnp
from jax import lax
from jax.experimental import pallas as pl
from jax.experimental.pallas import tpu as pltpu

_NCB = 4
_K = 512
_D = 64
_TT = 768
_A = 64

_HI = lax.Precision.HIGHEST
_BF = jnp.bfloat16


def _split3(x):
    hi = x.astype(_BF)
    r1 = x - hi.astype(jnp.float32)
    mid = r1.astype(_BF)
    lo = (r1 - mid.astype(jnp.float32)).astype(_BF)
    return hi, mid, lo


def _dot(a, b, dims):
    return lax.dot_general(a, b, (dims, ((), ())),
                           preferred_element_type=jnp.float32)


def _onehot_dot3(oh, parts, dims):
    h, m, l = parts
    ohb = oh.astype(_BF)
    return (_dot(ohb, h, dims) + _dot(ohb, m, dims)) + _dot(ohb, l, dims)


def _exact_dist(rt, wt, tb):
    diff = rt[:, :, None] - wt[:, None, :]
    sq = diff * diff
    x = sq.reshape(8, 8, tb, _K)
    x = x[:, 0:4] + x[:, 4:8]
    x = x[:, 0:2] + x[:, 2:4]
    x = x[:, 0] + x[:, 1]
    d = x[0]
    for g in range(1, 8):
        d = d + x[g]
    return d


def _first_min_idx(d, iota_k):
    m = jnp.min(d, axis=1, keepdims=True)
    return jnp.min(jnp.where(d == m, iota_k, _K), axis=1, keepdims=True), m


def _rvq_block(z_ref, cbt_ref, qsum_ref, inds_ref, loss_ref, idx_scr, zq_scr):
    i = pl.program_id(0)
    r = z_ref[...]
    lane_k = lax.broadcasted_iota(jnp.int32, (_TT, _K), 1)
    lane_ka = lax.broadcasted_iota(jnp.int32, (_A, _K), 1)
    lane128 = lax.broadcasted_iota(jnp.int32, (_TT, 128), 1)
    lane_a = lax.broadcasted_iota(jnp.int32, (_TT, _A), 1)
    row_t = lax.broadcasted_iota(jnp.int32, (_TT, _TT), 0)
    col_t = lax.broadcasted_iota(jnp.int32, (_TT, _TT), 1)
    ltri = (col_t < row_t).astype(jnp.float32)
    ind_tile = jnp.zeros((_TT, 128), jnp.int32)
    qacc = jnp.zeros((_TT, _D), jnp.float32)
    loss_val = jnp.float32(0.0)
    for c in range(_NCB):
        wt = cbt_ref[c]
        wparts = _split3(wt)
        rparts = _split3(r)
        s2 = jnp.sum(r * r, axis=1, keepdims=True)
        w2 = jnp.sum(wt * wt, axis=0, keepdims=True)
        cdims = ((1,), (0,))
        m2 = (_dot(rparts[0], wparts[0], cdims)
              + _dot(rparts[1], wparts[0], cdims)
              + _dot(rparts[0], wparts[1], cdims))
        shat = (s2 + w2) - (m2 + m2)
        idx, mhat = _first_min_idx(shat, lane_k)
        delta = s2 * jnp.float32(6e-6) + jnp.float32(3e-5)
        cnt = jnp.sum((shat <= mhat + delta).astype(jnp.int32),
                      axis=1, keepdims=True)
        amb = cnt > 1
        ambf = amb.astype(jnp.float32)
        namb = jnp.sum(ambf)
        rank = lax.dot_general(ltri, ambf, (((1,), (0,)), ((), ())),
                               preferred_element_type=jnp.float32)
        idx_scr[...] = jnp.broadcast_to(idx, (_TT, 128))
        oh = (lane_k == idx).astype(jnp.float32)
        zq0 = _onehot_dot3(oh, wparts, ((1,), (1,)))

        def _chunk(lo, wt=wt, wparts=wparts, rparts=rparts):
            in_rng = amb & (rank >= lo) & (rank < lo + _A)
            slot = (rank - lo).astype(jnp.int32)
            pt = (in_rng & (lane_a == slot)).astype(jnp.float32)
            rc = _onehot_dot3(pt, rparts, ((0,), (0,)))
            dex = _exact_dist(rc.T, wt, _A)
            iex, _ = _first_min_idx(dex, lane_ka)
            ohc = (lane_ka == iex).astype(jnp.float32)
            zqc = _onehot_dot3(ohc, wparts, ((1,), (1,)))
            zqf = _onehot_dot3(pt, _split3(zqc), ((1,), (0,)))
            iexf = iex.astype(jnp.float32)
            ih = iexf.astype(_BF)
            il = (iexf - ih.astype(jnp.float32)).astype(_BF)
            ptb = pt.astype(_BF)
            scat = (_dot(ptb, ih, ((1,), (0,)))
                    + _dot(ptb, il, ((1,), (0,))))
            return in_rng, zqf, scat

        in0, zqf0, scat0 = _chunk(jnp.float32(0.0))
        zq_scr[...] = jnp.where(jnp.broadcast_to(in0, (_TT, _D)), zqf0, zq0)
        idx_scr[...] = jnp.where(jnp.broadcast_to(in0, (_TT, 128)),
                                 jnp.broadcast_to(scat0.astype(jnp.int32),
                                                  (_TT, 128)),
                                 idx_scr[...])

        @pl.when(namb > jnp.float32(_A) + 0.5)
        def _(chunk=_chunk):
            nch = (namb.astype(jnp.int32) + (_A - 1)) // _A

            def _ovf(j, carry):
                in_j, zqf_j, scat_j = chunk((j * _A).astype(jnp.float32))
                zq_scr[...] = jnp.where(jnp.broadcast_to(in_j, (_TT, _D)),
                                        zqf_j, zq_scr[...])
                idx_scr[...] = jnp.where(
                    jnp.broadcast_to(in_j, (_TT, 128)),
                    jnp.broadcast_to(scat_j.astype(jnp.int32), (_TT, 128)),
                    idx_scr[...])
                return carry

            lax.fori_loop(1, nch, _ovf, 0)

        zq = zq_scr[...]
        idxf = idx_scr[:, 0:1]
        t = zq - r
        u = r + t
        loss_val = loss_val + jnp.sum(t * t)
        qacc = qacc + u
        r = r - u
        ind_tile = jnp.where(lane128 == c,
                             jnp.broadcast_to(idxf, (_TT, 128)), ind_tile)
    qsum_ref[...] = qacc
    inds_ref[...] = ind_tile

    @pl.when(i == 0)
    def _():
        loss_ref[...] = jnp.zeros_like(loss_ref)

    loss_ref[...] += jnp.full((8, 128), loss_val, jnp.float32)


def kernel(z, codebooks):
    B, N, D = z.shape
    T = B * N
    zf = z.reshape(T, D)
    cbt = jnp.transpose(codebooks, (0, 2, 1))
    qsum, indsw, lossw = pl.pallas_call(
        _rvq_block,
        grid=(T // _TT,),
        in_specs=[
            pl.BlockSpec((_TT, _D), lambda i: (i, 0)),
            pl.BlockSpec((_NCB, _D, _K), lambda i: (0, 0, 0)),
        ],
        out_specs=[
            pl.BlockSpec((_TT, _D), lambda i: (i, 0)),
            pl.BlockSpec((_TT, 128), lambda i: (i, 0)),
            pl.BlockSpec((8, 128), lambda i: (0, 0)),
        ],
        out_shape=[
            jax.ShapeDtypeStruct((T, _D), jnp.float32),
            jax.ShapeDtypeStruct((T, 128), jnp.int32),
            jax.ShapeDtypeStruct((8, 128), jnp.float32),
        ],
        scratch_shapes=[pltpu.VMEM((_TT, 128), jnp.int32),
                        pltpu.VMEM((_TT, _D), jnp.float32)],
    )(zf, cbt)
    quant_sum = qsum.reshape(B, N, D)
    inds = indsw[:, :_NCB].reshape(B, N, _NCB).transpose(0, 2, 1)
    total_loss = lossw[0, 0] * jnp.float32(2.0 / (B * N * D))
    return quant_sum, inds, total_loss


# speedup vs baseline: 1.6154x; 1.0934x over previous
import jax
import jax.numpy as jnp
from jax import lax
from jax.experimental import pallas as pl
from jax.experimental.pallas import tpu as pltpu

_NCB = 4
_K = 512
_D = 64
_TT = 1152
_A = 96

_HI = lax.Precision.HIGHEST
_BF = jnp.bfloat16


def _split3(x):
    hi = x.astype(_BF)
    r1 = x - hi.astype(jnp.float32)
    mid = r1.astype(_BF)
    lo = (r1 - mid.astype(jnp.float32)).astype(_BF)
    return hi, mid, lo


def _dot(a, b, dims):
    return lax.dot_general(a, b, (dims, ((), ())),
                           preferred_element_type=jnp.float32)


def _onehot_dot3(oh, parts, dims):
    h, m, l = parts
    ohb = oh.astype(_BF)
    return (_dot(ohb, h, dims) + _dot(ohb, m, dims)) + _dot(ohb, l, dims)


def _exact_dist(rt, wt, tb):
    diff = rt[:, :, None] - wt[:, None, :]
    sq = diff * diff
    x = sq.reshape(8, 8, tb, _K)
    x = x[:, 0:4] + x[:, 4:8]
    x = x[:, 0:2] + x[:, 2:4]
    x = x[:, 0] + x[:, 1]
    d = x[0]
    for g in range(1, 8):
        d = d + x[g]
    return d


def _first_min_idx(d, iota_k):
    m = jnp.min(d, axis=1, keepdims=True)
    return jnp.min(jnp.where(d == m, iota_k, _K), axis=1, keepdims=True), m


def _rvq_block(z_ref, cbt_ref, qsum_ref, inds_ref, loss_ref, idx_scr, zq_scr):
    i = pl.program_id(0)
    r = z_ref[...]
    lane_k = lax.broadcasted_iota(jnp.int32, (_TT, _K), 1)
    lane_ka = lax.broadcasted_iota(jnp.int32, (_A, _K), 1)
    lane128 = lax.broadcasted_iota(jnp.int32, (_TT, 128), 1)
    lane_a = lax.broadcasted_iota(jnp.int32, (_TT, _A), 1)
    row_t = lax.broadcasted_iota(jnp.int32, (_TT, _TT), 0)
    col_t = lax.broadcasted_iota(jnp.int32, (_TT, _TT), 1)
    ltri = (col_t < row_t).astype(jnp.float32)
    ind_tile = jnp.zeros((_TT, 128), jnp.int32)
    qacc = jnp.zeros((_TT, _D), jnp.float32)
    loss_val = jnp.float32(0.0)
    for c in range(_NCB):
        wt = cbt_ref[c]
        wparts = _split3(wt)
        rparts = _split3(r)
        s2 = jnp.sum(r * r, axis=1, keepdims=True)
        w2 = jnp.sum(wt * wt, axis=0, keepdims=True)
        cdims = ((1,), (0,))
        m2 = (_dot(rparts[0], wparts[0], cdims)
              + _dot(rparts[1], wparts[0], cdims)
              + _dot(rparts[0], wparts[1], cdims))
        shat = (s2 + w2) - (m2 + m2)
        idx, mhat = _first_min_idx(shat, lane_k)
        delta = s2 * jnp.float32(6e-6) + jnp.float32(3e-5)
        cnt = jnp.sum((shat <= mhat + delta).astype(jnp.int32),
                      axis=1, keepdims=True)
        amb = cnt > 1
        ambf = amb.astype(jnp.float32)
        namb = jnp.sum(ambf)
        rank = lax.dot_general(ltri, ambf, (((1,), (0,)), ((), ())),
                               preferred_element_type=jnp.float32)
        idx_scr[...] = jnp.broadcast_to(idx, (_TT, 128))
        oh = (lane_k == idx).astype(jnp.float32)
        zq0 = _onehot_dot3(oh, wparts, ((1,), (1,)))

        def _chunk(lo, wt=wt, wparts=wparts, rparts=rparts):
            in_rng = amb & (rank >= lo) & (rank < lo + _A)
            slot = (rank - lo).astype(jnp.int32)
            pt = (in_rng & (lane_a == slot)).astype(jnp.float32)
            rc = _onehot_dot3(pt, rparts, ((0,), (0,)))
            dex = _exact_dist(rc.T, wt, _A)
            iex, _ = _first_min_idx(dex, lane_ka)
            ohc = (lane_ka == iex).astype(jnp.float32)
            zqc = _onehot_dot3(ohc, wparts, ((1,), (1,)))
            zqf = _onehot_dot3(pt, _split3(zqc), ((1,), (0,)))
            iexf = iex.astype(jnp.float32)
            ih = iexf.astype(_BF)
            il = (iexf - ih.astype(jnp.float32)).astype(_BF)
            ptb = pt.astype(_BF)
            scat = (_dot(ptb, ih, ((1,), (0,)))
                    + _dot(ptb, il, ((1,), (0,))))
            return in_rng, zqf, scat

        in0, zqf0, scat0 = _chunk(jnp.float32(0.0))
        zq_scr[...] = jnp.where(jnp.broadcast_to(in0, (_TT, _D)), zqf0, zq0)
        idx_scr[...] = jnp.where(jnp.broadcast_to(in0, (_TT, 128)),
                                 jnp.broadcast_to(scat0.astype(jnp.int32),
                                                  (_TT, 128)),
                                 idx_scr[...])

        @pl.when(namb > jnp.float32(_A) + 0.5)
        def _(chunk=_chunk):
            nch = (namb.astype(jnp.int32) + (_A - 1)) // _A

            def _ovf(j, carry):
                in_j, zqf_j, scat_j = chunk((j * _A).astype(jnp.float32))
                zq_scr[...] = jnp.where(jnp.broadcast_to(in_j, (_TT, _D)),
                                        zqf_j, zq_scr[...])
                idx_scr[...] = jnp.where(
                    jnp.broadcast_to(in_j, (_TT, 128)),
                    jnp.broadcast_to(scat_j.astype(jnp.int32), (_TT, 128)),
                    idx_scr[...])
                return carry

            lax.fori_loop(1, nch, _ovf, 0)

        zq = zq_scr[...]
        idxf = idx_scr[:, 0:1]
        t = zq - r
        u = r + t
        loss_val = loss_val + jnp.sum(t * t)
        qacc = qacc + u
        r = r - u
        ind_tile = jnp.where(lane128 == c,
                             jnp.broadcast_to(idxf, (_TT, 128)), ind_tile)
    qsum_ref[...] = qacc
    inds_ref[...] = ind_tile

    @pl.when(i == 0)
    def _():
        loss_ref[...] = jnp.zeros_like(loss_ref)

    loss_ref[...] += jnp.full((8, 128), loss_val, jnp.float32)


def kernel(z, codebooks):
    B, N, D = z.shape
    T = B * N
    zf = z.reshape(T, D)
    cbt = jnp.transpose(codebooks, (0, 2, 1))
    qsum, indsw, lossw = pl.pallas_call(
        _rvq_block,
        grid=(T // _TT,),
        in_specs=[
            pl.BlockSpec((_TT, _D), lambda i: (i, 0)),
            pl.BlockSpec((_NCB, _D, _K), lambda i: (0, 0, 0)),
        ],
        out_specs=[
            pl.BlockSpec((_TT, _D), lambda i: (i, 0)),
            pl.BlockSpec((_TT, 128), lambda i: (i, 0)),
            pl.BlockSpec((8, 128), lambda i: (0, 0)),
        ],
        out_shape=[
            jax.ShapeDtypeStruct((T, _D), jnp.float32),
            jax.ShapeDtypeStruct((T, 128), jnp.int32),
            jax.ShapeDtypeStruct((8, 128), jnp.float32),
        ],
        scratch_shapes=[pltpu.VMEM((_TT, 128), jnp.int32),
                        pltpu.VMEM((_TT, _D), jnp.float32)],
    )(zf, cbt)
    quant_sum = qsum.reshape(B, N, D)
    inds = indsw[:, :_NCB].reshape(B, N, _NCB).transpose(0, 2, 1)
    total_loss = lossw[0, 0] * jnp.float32(2.0 / (B * N * D))
    return quant_sum, inds, total_loss
